# baseline (speedup 1.0000x reference)
import functools

import jax
import jax.numpy as jnp
from jax.experimental import pallas as pl

N = 8192
D = 64
BM = 256


def _fused_kernel(xs_ref, adj_ref, x_ref, w1t_ref, w2t_ref, b_ref, o_ref):
    neigh = jnp.dot(adj_ref[...], x_ref[...], preferred_element_type=jnp.float32)
    acc = jnp.dot(xs_ref[...], w1t_ref[...], preferred_element_type=jnp.float32)
    acc = acc + jnp.dot(neigh, w2t_ref[...], preferred_element_type=jnp.float32)
    o_ref[...] = jnp.maximum(acc + b_ref[...], 0.0)


@jax.jit
def kernel(x, adj_matrix, W, b):
    w1t = W[:, :D].T
    w2t = W[:, D:].T
    b2 = b.reshape(1, D)
    out = pl.pallas_call(
        _fused_kernel,
        grid=(N // BM,),
        in_specs=[
            pl.BlockSpec((BM, D), lambda i: (i, 0)),
            pl.BlockSpec((BM, N), lambda i: (i, 0)),
            pl.BlockSpec((N, D), lambda i: (0, 0)),
            pl.BlockSpec((D, D), lambda i: (0, 0)),
            pl.BlockSpec((D, D), lambda i: (0, 0)),
            pl.BlockSpec((1, D), lambda i: (0, 0)),
        ],
        out_specs=pl.BlockSpec((BM, D), lambda i: (i, 0)),
        out_shape=jax.ShapeDtypeStruct((N, D), jnp.float32),
    )(x, adj_matrix, x, w1t, w2t, b2)
    return out


# speedup vs baseline: 1.0249x; 1.0249x over previous
import functools

import jax
import jax.numpy as jnp
from jax.experimental import pallas as pl

N = 8192
D = 64
BM = 256


def _fused_kernel(xs_ref, adj_ref, x_ref, w1t_ref, w2t_ref, b_ref, o_ref):
    neigh = jnp.dot(
        adj_ref[...].astype(jnp.bfloat16),
        x_ref[...].astype(jnp.bfloat16),
        preferred_element_type=jnp.float32,
    )
    acc = jnp.dot(xs_ref[...], w1t_ref[...], preferred_element_type=jnp.float32)
    acc = acc + jnp.dot(neigh, w2t_ref[...], preferred_element_type=jnp.float32)
    o_ref[...] = jnp.maximum(acc + b_ref[...], 0.0)


@jax.jit
def kernel(x, adj_matrix, W, b):
    w1t = W[:, :D].T
    w2t = W[:, D:].T
    b2 = b.reshape(1, D)
    out = pl.pallas_call(
        _fused_kernel,
        grid=(N // BM,),
        in_specs=[
            pl.BlockSpec((BM, D), lambda i: (i, 0)),
            pl.BlockSpec((BM, N), lambda i: (i, 0)),
            pl.BlockSpec((N, D), lambda i: (0, 0)),
            pl.BlockSpec((D, D), lambda i: (0, 0)),
            pl.BlockSpec((D, D), lambda i: (0, 0)),
            pl.BlockSpec((1, D), lambda i: (0, 0)),
        ],
        out_specs=pl.BlockSpec((BM, D), lambda i: (i, 0)),
        out_shape=jax.ShapeDtypeStruct((N, D), jnp.float32),
    )(x, adj_matrix, x, w1t, w2t, b2)
    return out
